# baseline (speedup 1.0000x reference)
import functools

import jax
import jax.numpy as jnp
from jax import lax
from jax.experimental import pallas as pl
from jax.experimental.pallas import tpu as pltpu
from jax.experimental.pallas import tpu_sc as plsc

_NC = 2
_NS = 16
_NW = _NC * _NS

_B, _S, _D = 4, 2048, 1024
_N = _B * _S
_ROWS_W = _N // _NW
_CH = 16


def _sc_body(x_hbm, pe_hbm, o_hbm, xbuf, pebuf):
    c = lax.axis_index("c")
    s = lax.axis_index("s")
    wid = s * _NC + c
    row0 = wid * _ROWS_W
    pe_row0 = lax.rem(row0, _S)

    def chunk(i, _):
        xb = (row0 + i * _CH) * _D
        pb = (pe_row0 + i * _CH) * _D
        pltpu.sync_copy(x_hbm.at[pl.ds(xb, _CH * _D)], xbuf)
        pltpu.sync_copy(pe_hbm.at[pl.ds(pb, _CH * _D)], pebuf)

        def add16(j, _):
            sl = pl.ds(j * 16, 16)
            plsc.addupdate(xbuf.at[sl], pebuf[sl])
            return 0

        lax.fori_loop(0, _CH * _D // 16, add16, 0, unroll=8)
        pltpu.sync_copy(xbuf, o_hbm.at[pl.ds(xb, _CH * _D)])
        return 0

    lax.fori_loop(0, _ROWS_W // _CH, chunk, 0)


_sc_mesh = plsc.VectorSubcoreMesh(
    core_axis_name="c", subcore_axis_name="s", num_cores=_NC, num_subcores=_NS
)

_sc_add = pl.kernel(
    _sc_body,
    out_type=jax.ShapeDtypeStruct((_N * _D,), jnp.float32),
    mesh=_sc_mesh,
    scratch_types=[
        pltpu.VMEM((_CH * _D,), jnp.float32),
        pltpu.VMEM((_CH * _D,), jnp.float32),
    ],
)


def kernel(x, pe_weight):
    out = _sc_add(x.reshape(-1), pe_weight.reshape(-1))
    return out.reshape(x.shape)


# speedup vs baseline: 7.0673x; 7.0673x over previous
import jax
import jax.numpy as jnp
from jax.experimental import pallas as pl


_BS = 2048


def _add_body(x_ref, pe_ref, o_ref):
    o_ref[...] = x_ref[...] + pe_ref[...][None]


def kernel(x, pe_weight):
    B, S, D = x.shape
    grid = (S // _BS, B)
    return pl.pallas_call(
        _add_body,
        grid=grid,
        in_specs=[
            pl.BlockSpec((1, _BS, D), lambda s, b: (b, s, 0)),
            pl.BlockSpec((_BS, D), lambda s, b: (s, 0)),
        ],
        out_specs=pl.BlockSpec((1, _BS, D), lambda s, b: (b, s, 0)),
        out_shape=jax.ShapeDtypeStruct((B, S, D), x.dtype),
    )(x, pe_weight)


# speedup vs baseline: 7.0703x; 1.0004x over previous
import jax
import jax.numpy as jnp
from jax.experimental import pallas as pl
from jax.experimental.pallas import tpu as pltpu

_B, _S, _D = 4, 2048, 1024
_N = _B * _S
_C = 512
_NBUF = 4
_NCH = _N // _C
_PE_NCH = _S // _C


def _body(x_hbm, pe_hbm, o_hbm, xbuf, pebuf, obuf, insem, pesem, outsem):
    def xin(i):
        slot = i % _NBUF
        return pltpu.make_async_copy(
            x_hbm.at[pl.ds(i * _C, _C), :], xbuf.at[slot], insem.at[slot]
        )

    def pein(j):
        return pltpu.make_async_copy(
            pe_hbm.at[pl.ds(j * _C, _C), :], pebuf.at[pl.ds(j * _C, _C), :],
            pesem.at[j],
        )

    def oout(i):
        slot = i % _NBUF
        return pltpu.make_async_copy(
            obuf.at[slot], o_hbm.at[pl.ds(i * _C, _C), :], outsem.at[slot]
        )

    for j in range(_PE_NCH):
        pein(j).start()
    for i in range(_NBUF):
        xin(i).start()

    for i in range(_NCH):
        slot = i % _NBUF
        xin(i).wait()
        if i < _PE_NCH:
            pein(i).wait()
        if i >= _NBUF:
            oout(i - _NBUF).wait()
        poff = (i % _PE_NCH) * _C
        obuf[slot] = xbuf[slot] + pebuf[pl.ds(poff, _C), :]
        oout(i).start()
        if i + _NBUF < _NCH:
            xin(i + _NBUF).start()

    for i in range(_NCH - _NBUF, _NCH):
        oout(i).wait()


def kernel(x, pe_weight):
    out = pl.pallas_call(
        _body,
        in_specs=[
            pl.BlockSpec(memory_space=pltpu.MemorySpace.HBM),
            pl.BlockSpec(memory_space=pltpu.MemorySpace.HBM),
        ],
        out_specs=pl.BlockSpec(memory_space=pltpu.MemorySpace.HBM),
        out_shape=jax.ShapeDtypeStruct((_N, _D), x.dtype),
        scratch_shapes=[
            pltpu.VMEM((_NBUF, _C, _D), jnp.float32),
            pltpu.VMEM((_S, _D), jnp.float32),
            pltpu.VMEM((_NBUF, _C, _D), jnp.float32),
            pltpu.SemaphoreType.DMA((_NBUF,)),
            pltpu.SemaphoreType.DMA((_PE_NCH,)),
            pltpu.SemaphoreType.DMA((_NBUF,)),
        ],
    )(x.reshape(_N, _D), pe_weight)
    return out.reshape(x.shape)
